# Initial kernel scaffold; baseline (speedup 1.0000x reference)
import math

import jax
import jax.numpy as jnp
import numpy as np
from jax import lax
from jax.experimental import pallas as pl
from jax.experimental.pallas import tpu as pltpu
from jax.experimental.pallas import tpu_sc as plsc

_CLIP_LIMIT = 0.8
_GRID = (8, 8)
_NUM_BINS = 256

_NUM_CORES = 2
_NUM_SUBCORES = 16
_NWORKERS = _NUM_CORES * _NUM_SUBCORES
_L = 16


def _sc_mesh():
    return plsc.VectorSubcoreMesh(
        core_axis_name="c", subcore_axis_name="s",
        num_cores=_NUM_CORES, num_subcores=_NUM_SUBCORES)


def _make_hist_kernel(BC, th, tw, gh, gw):
    T = BC * gh * gw
    tiles_per_worker = T // _NWORKERS
    row = _L * _NUM_BINS

    def body(x_hbm, hist_hbm, tile_v, hist_v):
        wid = lax.axis_index("s") * _NUM_CORES + lax.axis_index("c")
        lane_off = lax.iota(jnp.int32, _L) * _NUM_BINS
        ones = jnp.ones((_L,), jnp.float32)
        zeros = jnp.zeros((_L,), jnp.float32)

        @pl.loop(0, tiles_per_worker)
        def _(j):
            t = wid * tiles_per_worker + j
            bc = t // (gh * gw)
            rem = t % (gh * gw)
            gy = rem // gw
            gx = rem % gw
            pltpu.sync_copy(
                x_hbm.at[bc, pl.ds(gy * th, th), pl.ds(gx * tw, tw)], tile_v)

            @pl.loop(0, row // _L)
            def _(i):
                hist_v[pl.ds(i * _L, _L)] = zeros

            @pl.loop(0, th)
            def _(r):
                for cc in range(tw // _L):
                    v = tile_v[r, pl.ds(cc * _L, _L)]
                    b = (v * float(_NUM_BINS)).astype(jnp.int32)
                    b = jnp.minimum(jnp.maximum(b, 0), _NUM_BINS - 1)
                    plsc.addupdate_scatter(hist_v, [lane_off + b], ones)

            pltpu.sync_copy(hist_v, hist_hbm.at[t])

    return pl.kernel(
        body,
        out_type=jax.ShapeDtypeStruct((T, row), jnp.float32),
        mesh=_sc_mesh(),
        scratch_types=[
            pltpu.VMEM((th, tw), jnp.float32),
            pltpu.VMEM((row,), jnp.float32),
        ],
    )


def _make_luts_kernel(T, pixels):
    max_val = float(max(_CLIP_LIMIT * pixels // _NUM_BINS, 1))
    lut_scale = (_NUM_BINS - 1) / pixels
    rows_blk = 256
    nblk = T // rows_blk

    def body(h_ref, o_ref):
        hb = h_ref[...]
        h = hb[:, 0:_NUM_BINS]
        for l in range(1, _L):
            h = h + hb[:, l * _NUM_BINS:(l + 1) * _NUM_BINS]
        h = jnp.minimum(h, max_val)
        clipped = float(pixels) - jnp.sum(h, axis=1, keepdims=True)
        redist = jnp.floor(clipped * (1.0 / _NUM_BINS))
        residual = clipped - redist * float(_NUM_BINS)
        col = lax.broadcasted_iota(jnp.float32, h.shape, 1)
        h = h + redist + (col < residual).astype(jnp.float32)
        for k in (1, 2, 4, 8, 16, 32, 64, 128):
            z = jnp.zeros((h.shape[0], k), jnp.float32)
            h = h + jnp.concatenate([z, h[:, :_NUM_BINS - k]], axis=1)
        o_ref[...] = jnp.floor(
            jnp.clip(h * lut_scale, 0.0, float(_NUM_BINS - 1)))

    return pl.pallas_call(
        body,
        grid=(nblk,),
        in_specs=[pl.BlockSpec((rows_blk, _L * _NUM_BINS), lambda i: (i, 0))],
        out_specs=pl.BlockSpec((rows_blk, _NUM_BINS), lambda i: (i, 0)),
        out_shape=jax.ShapeDtypeStruct((T, _NUM_BINS), jnp.float32),
    )


def _make_apply_kernel(BC, H, W, th, tw, gh, gw):
    rows_blk = 32
    ntasks = BC * (H // rows_blk)
    tasks_per_worker = ntasks // _NWORKERS
    blocks_per_img = H // rows_blk
    slab = gh * gw * _NUM_BINS

    def body(x_hbm, luts_hbm, y0e_hbm, y1e_hbm, wye_hbm,
             x0c_hbm, x1c_hbm, wxc_hbm, out_hbm,
             xv, lutv, ov, y0v, y1v, wyv, x0v, x1v, wxv):
        wid = lax.axis_index("s") * _NUM_CORES + lax.axis_index("c")
        pltpu.sync_copy(y0e_hbm, y0v)
        pltpu.sync_copy(y1e_hbm, y1v)
        pltpu.sync_copy(wye_hbm, wyv)
        pltpu.sync_copy(x0c_hbm, x0v)
        pltpu.sync_copy(x1c_hbm, x1v)
        pltpu.sync_copy(wxc_hbm, wxv)

        @pl.loop(0, tasks_per_worker)
        def _(j):
            t = wid * tasks_per_worker + j
            bc = t // blocks_per_img
            r0 = (t % blocks_per_img) * rows_blk
            pltpu.sync_copy(luts_hbm.at[bc], lutv)
            pltpu.sync_copy(x_hbm.at[bc, pl.ds(r0, rows_blk)], xv)

            @pl.loop(0, rows_blk)
            def _(rl):
                rg = r0 + rl
                yo0 = y0v[rg]
                yo1 = y1v[rg]
                wyr = wyv[rg]

                @pl.loop(0, W // _L)
                def _(cc):
                    co = cc * _L
                    xx = xv[rl, pl.ds(co, _L)]
                    xo0 = x0v[pl.ds(co, _L)]
                    xo1 = x1v[pl.ds(co, _L)]
                    wxr = wxv[pl.ds(co, _L)]
                    b = (xx * float(_NUM_BINS - 1)).astype(jnp.int32)
                    b = jnp.minimum(jnp.maximum(b, 0), _NUM_BINS - 1)
                    v00 = plsc.load_gather(lutv, [b + yo0 + xo0])
                    v01 = plsc.load_gather(lutv, [b + yo0 + xo1])
                    v10 = plsc.load_gather(lutv, [b + yo1 + xo0])
                    v11 = plsc.load_gather(lutv, [b + yo1 + xo1])
                    top = v00 + wxr * (v01 - v00)
                    bot = v10 + wxr * (v11 - v10)
                    o = top + wyr * (bot - top)
                    ov[rl, pl.ds(co, _L)] = o * (1.0 / (_NUM_BINS - 1))

            pltpu.sync_copy(ov, out_hbm.at[bc, pl.ds(r0, rows_blk)])

    return pl.kernel(
        body,
        out_type=jax.ShapeDtypeStruct((BC, H, W), jnp.float32),
        mesh=_sc_mesh(),
        scratch_types=[
            pltpu.VMEM((rows_blk, W), jnp.float32),
            pltpu.VMEM((slab,), jnp.float32),
            pltpu.VMEM((rows_blk, W), jnp.float32),
            pltpu.VMEM((H, _L), jnp.int32),
            pltpu.VMEM((H, _L), jnp.int32),
            pltpu.VMEM((H, _L), jnp.float32),
            pltpu.VMEM((W,), jnp.int32),
            pltpu.VMEM((W,), jnp.int32),
            pltpu.VMEM((W,), jnp.float32),
        ],
    )


def kernel(x):
    B, C, H, W = x.shape
    gh, gw = _GRID
    th = math.ceil(H / gh)
    tw = math.ceil(W / gw)
    assert th * gh == H and tw * gw == W, "padding path not needed here"
    BC = B * C
    pixels = th * tw

    fy = (np.arange(H, dtype=np.float64) + 0.5) / th - 0.5
    fx = (np.arange(W, dtype=np.float64) + 0.5) / tw - 0.5
    ty = np.clip(fy, 0.0, gh - 1.0).astype(np.float32)
    tx = np.clip(fx, 0.0, gw - 1.0).astype(np.float32)
    y0 = np.floor(ty).astype(np.int32)
    x0 = np.floor(tx).astype(np.int32)
    y1 = np.minimum(y0 + 1, gh - 1)
    x1 = np.minimum(x0 + 1, gw - 1)
    wy = (ty - y0).astype(np.float32)
    wx = (tx - x0).astype(np.float32)
    y0e = jnp.asarray(np.repeat((y0 * gw * _NUM_BINS)[:, None], _L, axis=1))
    y1e = jnp.asarray(np.repeat((y1 * gw * _NUM_BINS)[:, None], _L, axis=1))
    wye = jnp.asarray(np.repeat(wy[:, None], _L, axis=1))
    x0c = jnp.asarray(x0 * _NUM_BINS)
    x1c = jnp.asarray(x1 * _NUM_BINS)
    wxc = jnp.asarray(wx)

    x3 = x.reshape(BC, H, W)
    hist = _make_hist_kernel(BC, th, tw, gh, gw)(x3)
    luts = _make_luts_kernel(BC * gh * gw, pixels)(hist)
    luts2 = luts.reshape(BC, gh * gw * _NUM_BINS)
    out3 = _make_apply_kernel(BC, H, W, th, tw, gh, gw)(
        x3, luts2, y0e, y1e, wye, x0c, x1c, wxc)
    return out3.reshape(B, C, H, W)


# speedup vs baseline: 2964.5833x; 2964.5833x over previous
import dataclasses
import math

import jax
import jax.numpy as jnp
import numpy as np
from jax import lax
from jax.experimental import pallas as pl
from jax.experimental.pallas import tpu as pltpu
from jax.experimental.pallas import tpu_sc as plsc

_CLIP_LIMIT = 0.8
_GRID = (8, 8)
_NUM_BINS = 256

_NUM_CORES = 2
_NUM_SUBCORES = 16
_NWORKERS = _NUM_CORES * _NUM_SUBCORES
_L = 16


def _sc_mesh():
    return plsc.VectorSubcoreMesh(
        core_axis_name="c", subcore_axis_name="s",
        num_cores=_NUM_CORES, num_subcores=_NUM_SUBCORES)


def _sc_params():
    cp = pltpu.CompilerParams()
    if "needs_layout_passes" in pltpu.CompilerParams.__dataclass_fields__:
        cp = dataclasses.replace(cp, needs_layout_passes=False)
    return cp


def _make_hist_kernel(BC, th, tw, gh, gw, W):
    T = BC * gh * gw
    strips = BC * gh
    strips_per_worker = strips // _NWORKERS
    assert strips_per_worker % 2 == 0
    row = _L * _NUM_BINS
    hh = th // 2

    def body(x_hbm, hist_hbm, hb0, hb1, hv0, hv1, si0, si1, so0, so1):
        wid = lax.axis_index("s") * _NUM_CORES + lax.axis_index("c")
        lane_off = lax.iota(jnp.int32, _L)
        ones = jnp.ones((_L,), jnp.float32)
        zeros = jnp.zeros((_L,), jnp.float32)
        hbs = (hb0, hb1)
        hvs = (hv0, hv1)
        sis = (si0, si1)
        sos = (so0, so1)

        def strip_coords(s):
            return s // gh, s % gh

        def start_half(s, half):
            bc, gy = strip_coords(s)
            pltpu.async_copy(
                x_hbm.at[bc, pl.ds(gy * th + half * hh, hh)], hbs[half],
                sis[half])

        def wait_half(s, half):
            bc, gy = strip_coords(s)
            pltpu.make_async_copy(
                x_hbm.at[bc, pl.ds(gy * th + half * hh, hh)], hbs[half],
                sis[half]).wait()

        def scatter_half(half, hv):
            @pl.loop(0, W // _L)
            def _(cc):
                gx = cc * _L // tw
                gxv = lax.broadcast_in_dim(gx, (_L,), ())

                @plsc.parallel_loop(0, hh, unroll=8)
                def _(r):
                    v = hbs[half][r, pl.ds(cc * _L, _L)]
                    b = (v * float(_NUM_BINS)).astype(jnp.int32)
                    plsc.addupdate_scatter(hv, [gxv, lane_off + b * _L], ones)

        start_half(wid * strips_per_worker, 0)

        @pl.loop(0, strips_per_worker // 2)
        def _(ss):
            for ph in range(2):
                s = wid * strips_per_worker + ss * 2 + ph
                hv = hvs[ph]
                start_half(s, 1)
                @pl.when(ss >= 1)
                def _():
                    pltpu.make_async_copy(
                        hv, hist_hbm.at[pl.ds(0, gw)], sos[ph]).wait()
                for gx in range(gw):
                    @plsc.parallel_loop(0, row // _L, unroll=4)
                    def _(i, gx=gx):
                        hv[gx, pl.ds(i * _L, _L)] = zeros
                wait_half(s, 0)
                scatter_half(0, hv)
                if ph == 0:
                    start_half(s + 1, 0)
                else:
                    @pl.when(ss < strips_per_worker // 2 - 1)
                    def _():
                        start_half(s + 1, 0)
                wait_half(s, 1)
                scatter_half(1, hv)
                pltpu.async_copy(hv, hist_hbm.at[pl.ds(s * gw, gw)], sos[ph])

        for ph in range(2):
            pltpu.make_async_copy(
                hvs[ph], hist_hbm.at[pl.ds(0, gw)], sos[ph]).wait()

    return pl.kernel(
        body,
        out_type=jax.ShapeDtypeStruct((T, row), jnp.float32),
        mesh=_sc_mesh(),
        compiler_params=_sc_params(),
        scratch_types=[
            pltpu.VMEM((hh, W), jnp.float32),
            pltpu.VMEM((hh, W), jnp.float32),
            pltpu.VMEM((gw, row), jnp.float32),
            pltpu.VMEM((gw, row), jnp.float32),
            pltpu.SemaphoreType.DMA,
            pltpu.SemaphoreType.DMA,
            pltpu.SemaphoreType.DMA,
            pltpu.SemaphoreType.DMA,
        ],
    )


def _make_luts_kernel(T, pixels, gh, gw):
    max_val = float(max(_CLIP_LIMIT * pixels // _NUM_BINS, 1))
    lut_scale = (_NUM_BINS - 1) / pixels
    rows_blk = 256
    nblk = T // rows_blk
    tiles_per_img = gh * gw
    assert rows_blk % tiles_per_img == 0

    def body(h_ref, o_ref):
        hb = h_ref[...]
        rows = lax.broadcasted_iota(jnp.int32, (_L * _NUM_BINS, _NUM_BINS), 0)
        cols = lax.broadcasted_iota(jnp.int32, (_L * _NUM_BINS, _NUM_BINS), 1)
        sel = (rows // _L == cols).astype(jnp.float32)
        h = jax.lax.dot_general(
            hb, sel, (((1,), (0,)), ((), ())),
            preferred_element_type=jnp.float32)
        h = jnp.minimum(h, max_val)
        clipped = float(pixels) - jnp.sum(h, axis=1, keepdims=True)
        redist = jnp.floor(clipped * (1.0 / _NUM_BINS))
        residual = clipped - redist * float(_NUM_BINS)
        col = lax.broadcasted_iota(jnp.int32, h.shape, 1).astype(jnp.float32)
        h = h + redist + (col < residual).astype(jnp.float32)
        for k in (1, 2, 4, 8, 16, 32, 64, 128):
            z = jnp.zeros((h.shape[0], k), jnp.float32)
            h = h + jnp.concatenate([z, h[:, :_NUM_BINS - k]], axis=1)
        lut = jnp.floor(jnp.clip(h * lut_scale, 0.0, float(_NUM_BINS - 1)))
        lut = lut * (1.0 / float(_NUM_BINS - 1))
        bots = []
        for g in range(rows_blk // tiles_per_img):
            grp = lut[g * tiles_per_img:(g + 1) * tiles_per_img]
            bots.append(jnp.concatenate(
                [grp[gw:], grp[tiles_per_img - gw:]], axis=0))
        bot = jnp.concatenate(bots, axis=0)
        t16 = lax.bitcast_convert_type(lut.astype(jnp.bfloat16), jnp.uint16)
        b16 = lax.bitcast_convert_type(bot.astype(jnp.bfloat16), jnp.uint16)
        packed = (t16.astype(jnp.uint32) << 16) | b16.astype(jnp.uint32)
        o_ref[...] = lax.bitcast_convert_type(packed, jnp.int32)

    return pl.pallas_call(
        body,
        grid=(nblk,),
        in_specs=[pl.BlockSpec((rows_blk, _L * _NUM_BINS), lambda i: (i, 0))],
        out_specs=pl.BlockSpec((rows_blk, _NUM_BINS), lambda i: (i, 0)),
        out_shape=jax.ShapeDtypeStruct((T, _NUM_BINS), jnp.int32),
    )


def _make_apply_kernel(BC, H, W, th, tw, gh, gw):
    rows_blk = 32
    ntasks = BC * (H // rows_blk)
    tasks_per_worker = ntasks // _NWORKERS
    assert tasks_per_worker % 2 == 0
    blocks_per_img = H // rows_blk
    slab = gh * gw * _NUM_BINS

    def body(x_hbm, luts_hbm, y0e_hbm, dye_hbm, wye_hbm,
             x0c_hbm, dxc_hbm, wxc_hbm, out_hbm,
             xv0, xv1, lutv, ov0, ov1,
             y0v, dyv, wyv, x0v, dxv, wxv,
             sx0, sx1, sl, so0, so1):
        wid = lax.axis_index("s") * _NUM_CORES + lax.axis_index("c")
        pltpu.sync_copy(y0e_hbm, y0v)
        pltpu.sync_copy(dye_hbm, dyv)
        pltpu.sync_copy(wye_hbm, wyv)
        pltpu.sync_copy(x0c_hbm, x0v)
        pltpu.sync_copy(dxc_hbm, dxv)
        pltpu.sync_copy(wxc_hbm, wxv)

        xvs = (xv0, xv1)
        ovs = (ov0, ov1)
        sxs = (sx0, sx1)
        sos = (so0, so1)

        def task_coords(t):
            return t // blocks_per_img, (t % blocks_per_img) * rows_blk

        def start_in(t, ph):
            bc, r0 = task_coords(t)
            pltpu.async_copy(x_hbm.at[bc, pl.ds(r0, rows_blk)], xvs[ph],
                             sxs[ph])

        start_in(wid * tasks_per_worker, 0)

        @pl.loop(0, tasks_per_worker // 2, init_carry=jnp.int32(-1))
        def _(jj, bcp):
            for ph in range(2):
                jt = jj * 2 + ph
                t = wid * tasks_per_worker + jt
                bc, r0 = task_coords(t)
                if ph == 0:
                    start_in(t + 1, 1)
                else:
                    @pl.when(jj < tasks_per_worker // 2 - 1)
                    def _():
                        start_in(t + 1, 0)
                @pl.when(bc != bcp)
                def _():
                    pltpu.async_copy(luts_hbm.at[bc], lutv, sl).wait()
                bcp = bc
                pltpu.make_async_copy(
                    x_hbm.at[bc, pl.ds(r0, rows_blk)], xvs[ph],
                    sxs[ph]).wait()
                @pl.when(jj >= 1)
                def _():
                    pltpu.make_async_copy(
                        ovs[ph], out_hbm.at[bc, pl.ds(r0, rows_blk)],
                        sos[ph]).wait()

                xvp, lutvp, ovp = xvs[ph], lutv, ovs[ph]

                @pl.loop(0, rows_blk)
                def _(rl):
                    rg = r0 + rl
                    yo0 = y0v[pl.ds(rg * _L, _L)]
                    wyr = wyv[pl.ds(rg * _L, _L)]

                    @plsc.parallel_loop(0, W // _L, unroll=8)
                    def _(cc):
                        co = cc * _L
                        xx = xvp[rl, pl.ds(co, _L)]
                        xo0 = x0v[pl.ds(co, _L)]
                        dxr = dxv[pl.ds(co, _L)]
                        wxr = wxv[pl.ds(co, _L)]
                        b = (xx * float(_NUM_BINS - 1)).astype(jnp.int32)
                        iA = b + yo0 + xo0
                        iB = iA + dxr
                        gA = plsc.load_gather(lutvp, [iA])
                        gB = plsc.load_gather(lutvp, [iB])
                        hi = jnp.int32(-65536)
                        v00 = plsc.bitcast(gA & hi, jnp.float32)
                        v10 = plsc.bitcast(gA << 16, jnp.float32)
                        v01 = plsc.bitcast(gB & hi, jnp.float32)
                        v11 = plsc.bitcast(gB << 16, jnp.float32)
                        top = v00 + wxr * (v01 - v00)
                        bot = v10 + wxr * (v11 - v10)
                        o = top + wyr * (bot - top)
                        ovp[rl, pl.ds(co, _L)] = o

                pltpu.async_copy(
                    ovp, out_hbm.at[bc, pl.ds(r0, rows_blk)], sos[ph])
            return bcp

        for ph in range(2):
            pltpu.make_async_copy(
                ovs[ph], out_hbm.at[0, pl.ds(0, rows_blk)], sos[ph]).wait()

    return pl.kernel(
        body,
        out_type=jax.ShapeDtypeStruct((BC, H, W), jnp.float32),
        mesh=_sc_mesh(),
        compiler_params=_sc_params(),
        scratch_types=[
            pltpu.VMEM((rows_blk, W), jnp.float32),
            pltpu.VMEM((rows_blk, W), jnp.float32),
            pltpu.VMEM((slab,), jnp.int32),
            pltpu.VMEM((rows_blk, W), jnp.float32),
            pltpu.VMEM((rows_blk, W), jnp.float32),
            pltpu.VMEM((H * _L,), jnp.int32),
            pltpu.VMEM((H * _L,), jnp.int32),
            pltpu.VMEM((H * _L,), jnp.float32),
            pltpu.VMEM((W,), jnp.int32),
            pltpu.VMEM((W,), jnp.int32),
            pltpu.VMEM((W,), jnp.float32),
            pltpu.SemaphoreType.DMA,
            pltpu.SemaphoreType.DMA,
            pltpu.SemaphoreType.DMA,
            pltpu.SemaphoreType.DMA,
            pltpu.SemaphoreType.DMA,
        ],
    )


def kernel(x):
    B, C, H, W = x.shape
    gh, gw = _GRID
    th = math.ceil(H / gh)
    tw = math.ceil(W / gw)
    assert th * gh == H and tw * gw == W, "padding path not needed here"
    BC = B * C
    pixels = th * tw

    fy = (np.arange(H, dtype=np.float64) + 0.5) / th - 0.5
    fx = (np.arange(W, dtype=np.float64) + 0.5) / tw - 0.5
    ty = np.clip(fy, 0.0, gh - 1.0).astype(np.float32)
    tx = np.clip(fx, 0.0, gw - 1.0).astype(np.float32)
    y0 = np.floor(ty).astype(np.int32)
    x0 = np.floor(tx).astype(np.int32)
    y1 = np.minimum(y0 + 1, gh - 1)
    x1 = np.minimum(x0 + 1, gw - 1)
    wy = (ty - y0).astype(np.float32)
    wx = (tx - x0).astype(np.float32)
    y0e = jnp.asarray(
        np.repeat((y0 * gw * _NUM_BINS)[:, None], _L, axis=1).reshape(-1))
    dye = jnp.asarray(
        np.repeat(((y1 - y0) * gw * _NUM_BINS)[:, None], _L,
                  axis=1).reshape(-1))
    wye = jnp.asarray(np.repeat(wy[:, None], _L, axis=1).reshape(-1))
    x0c = jnp.asarray(x0 * _NUM_BINS)
    dxc = jnp.asarray((x1 - x0) * _NUM_BINS)
    wxc = jnp.asarray(wx)

    x3 = x.reshape(BC, H, W)
    hist = _make_hist_kernel(BC, th, tw, gh, gw, W)(x3)
    luts = _make_luts_kernel(BC * gh * gw, pixels, gh, gw)(hist)
    luts2 = luts.reshape(BC, gh * gw * _NUM_BINS)
    out3 = _make_apply_kernel(BC, H, W, th, tw, gh, gw)(
        x3, luts2, y0e, dye, wye, x0c, dxc, wxc)
    return out3.reshape(B, C, H, W)
